# baseline (speedup 1.0000x reference)
import functools

import jax
import jax.numpy as jnp
from jax.experimental import pallas as pl
from jax.experimental.pallas import tpu as pltpu

_BM = 400


def _gcn_kernel(x_ref, w1_ref, w2_ref, adj_ref, o_ref,
                s1_ref, s2_ref, *, nb, bm):
    i = pl.program_id(0)
    dn = (((1,), (0,)), ((), ()))

    @pl.when(i == 0)
    def _():
        s1_ref[...] = jnp.dot(x_ref[...], w1_ref[...],
                              preferred_element_type=jnp.float32
                              ).astype(jnp.bfloat16)

    @pl.when(i < nb)
    def _():
        t = jax.lax.dot_general(adj_ref[...], s1_ref[...], dn,
                                preferred_element_type=jnp.float32)
        h = jnp.maximum(t, 0.0)
        s2_ref[pl.ds(i * bm, bm), :] = jnp.dot(
            h, w2_ref[...], preferred_element_type=jnp.float32
            ).astype(jnp.bfloat16)

    @pl.when(i >= nb)
    def _():
        o_ref[...] = jax.lax.dot_general(adj_ref[...], s2_ref[...], dn,
                                         preferred_element_type=jnp.float32)


def kernel(x, adj, W1, W2):
    n, nfeat = x.shape
    nhid = W1.shape[1]
    nclass = W2.shape[1]
    bm = _BM
    nb = n // bm

    once = pl.Buffered(buffer_count=1)
    return pl.pallas_call(
        functools.partial(_gcn_kernel, nb=nb, bm=bm),
        grid=(2 * nb,),
        in_specs=[
            pl.BlockSpec((n, nfeat), lambda i: (0, 0), pipeline_mode=once),
            pl.BlockSpec((nfeat, nhid), lambda i: (0, 0), pipeline_mode=once),
            pl.BlockSpec((nhid, nclass), lambda i: (0, 0), pipeline_mode=once),
            pl.BlockSpec((bm, n), lambda i: (jax.lax.rem(i, nb), 0)),
        ],
        out_specs=pl.BlockSpec((bm, nclass),
                               lambda i: (jnp.maximum(i - nb, 0), 0)),
        out_shape=jax.ShapeDtypeStruct((n, nclass), jnp.float32),
        scratch_shapes=[
            pltpu.VMEM((n, nhid), jnp.bfloat16),
            pltpu.VMEM((n, nclass), jnp.bfloat16),
        ],
    )(x, W1, W2, adj)
